# baseline (speedup 1.0000x reference)
import functools

import jax
import jax.numpy as jnp
from jax import lax
from jax.experimental import pallas as pl
from jax.experimental.pallas import tpu as pltpu
from jax.experimental.pallas import tpu_sc as plsc

_B = 1
_N = 16384
_PATCH = 1024
_GRID = 0.02
_ENC_DEPTHS = (2, 2, 2, 4, 2)
_ENC_CH = (32, 64, 128, 256, 384)
_ENC_H = (2, 4, 8, 16, 24)
_DEC_DEPTHS = (2, 2, 2, 2)
_DEC_CH = (64, 64, 128, 256)
_DEC_H = (4, 4, 8, 16)


def _split3(a):
    a = a & jnp.uint32(0x3FF)
    a = (a | (a << 16)) & jnp.uint32(0x030000FF)
    a = (a | (a << 8)) & jnp.uint32(0x0300F00F)
    a = (a | (a << 4)) & jnp.uint32(0x030C30C3)
    a = (a | (a << 2)) & jnp.uint32(0x09249249)
    return a


def _morton(coord, grid):
    g = jnp.floor(coord / grid).astype(jnp.uint32)
    return _split3(g[:, 0]) | (_split3(g[:, 1]) << 1) | (_split3(g[:, 2]) << 2)


def _all_orders(flat):
    coord = flat
    orders = []
    for s in range(5):
        code = _morton(coord, _GRID * (2 ** s))
        order = jnp.argsort(code)
        coord = coord[order]
        orders.append(order)
        if s < 4:
            coord = jnp.mean(coord.reshape(-1, 2, 3), axis=1)
    return orders


def _inv_perm(order):
    return jnp.argsort(order)


_SC_NC = 2
_SC_NS = 16
_SC_NW = _SC_NC * _SC_NS


def _sc_gather(x, idx):
    M, C = x.shape
    if C % 128 != 0:
        return x[idx]
    bpw = M // _SC_NW
    chunk = min(bpw, 128)
    nchunk = bpw // chunk
    mesh = plsc.VectorSubcoreMesh(core_axis_name="c", subcore_axis_name="s")

    @functools.partial(
        pl.kernel, mesh=mesh,
        out_type=jax.ShapeDtypeStruct((M, C), jnp.float32),
        scratch_types=[pltpu.VMEM((bpw,), jnp.int32),
                       pltpu.VMEM((bpw, C), jnp.float32),
                       pltpu.SemaphoreType.DMA])
    def gk(x_hbm, idx_hbm, out_hbm, idx_v, rows_v, sem):
        wid = lax.axis_index("s") * _SC_NC + lax.axis_index("c")
        base = wid * bpw
        pltpu.sync_copy(idx_hbm.at[pl.ds(base, bpw)], idx_v)
        copies = [
            pltpu.async_copy(
                x_hbm.at[idx_v.at[pl.ds(j * chunk, chunk)]],
                rows_v.at[pl.ds(j * chunk, chunk)], sem)
            for j in range(nchunk)
        ]
        for c in copies:
            c.wait()
        pltpu.sync_copy(rows_v, out_hbm.at[pl.ds(base, bpw)])

    return gk(x, idx.astype(jnp.int32))


def _ln(x, s, b):
    mu = jnp.mean(x, axis=-1, keepdims=True)
    var = jnp.mean((x - mu) ** 2, axis=-1, keepdims=True)
    return (x - mu) / jnp.sqrt(var + 1e-5) * s + b


def _attn_body(nheads, x_ref, ln1s, ln1b, qkvw, qkvb, projw, projb,
               ln2s, ln2b, w1, b1, w2, b2, o_ref):
    x = x_ref[...]
    C = x.shape[1]
    d = C // nheads
    h = _ln(x, ln1s[...], ln1b[...])
    qkv = jnp.dot(h, qkvw[...], preferred_element_type=jnp.float32) + qkvb[...]
    scale = d ** -0.5
    outs = []
    qkv16 = qkv.astype(jnp.bfloat16)
    ones = jnp.ones((x.shape[0], 1), jnp.bfloat16)
    scale = scale * 1.4426950408889634
    for hd in range(nheads):
        q = (qkv[:, hd * d:(hd + 1) * d] * scale).astype(jnp.bfloat16)
        k = qkv16[:, C + hd * d:C + (hd + 1) * d]
        v = qkv16[:, 2 * C + hd * d:2 * C + (hd + 1) * d]
        s = jax.lax.dot_general(q, k, (((1,), (1,)), ((), ())),
                                preferred_element_type=jnp.float32)
        e = jnp.exp2(s.astype(jnp.bfloat16))
        ov = jnp.dot(e, jnp.concatenate([v, ones], axis=1),
                     preferred_element_type=jnp.float32)
        outs.append(ov[:, :d] / ov[:, d:d + 1])
    o = jnp.concatenate(outs, axis=1)
    x = x + jnp.dot(o, projw[...], preferred_element_type=jnp.float32) + projb[...]
    h = _ln(x, ln2s[...], ln2b[...])
    h = jax.nn.gelu(jnp.dot(h, w1[...], preferred_element_type=jnp.float32) + b1[...])
    h = jnp.dot(h, w2[...], preferred_element_type=jnp.float32) + b2[...]
    o_ref[...] = x + h


def _attn_block(x, p, nheads):
    M, C = x.shape
    ws = [p['ln1_s'][None, :], p['ln1_b'][None, :], p['qkv_w'], p['qkv_b'][None, :],
          p['proj_w'], p['proj_b'][None, :], p['ln2_s'][None, :], p['ln2_b'][None, :],
          p['mlp_w1'], p['mlp_b1'][None, :], p['mlp_w2'], p['mlp_b2'][None, :]]
    grid = (M // _PATCH,)
    in_specs = [pl.BlockSpec((_PATCH, C), lambda i: (i, 0))]
    in_specs += [pl.BlockSpec(w.shape, lambda i: (0, 0)) for w in ws]
    return pl.pallas_call(
        functools.partial(_attn_body, nheads),
        grid=grid,
        in_specs=in_specs,
        out_specs=pl.BlockSpec((_PATCH, C), lambda i: (i, 0)),
        out_shape=jax.ShapeDtypeStruct((M, C), jnp.float32),
        compiler_params=pltpu.CompilerParams(
            dimension_semantics=("parallel",)),
    )(x, *ws)


def _down_body(x2_ref, w_ref, b_ref, o_ref):
    C = w_ref.shape[0]
    x2 = x2_ref[...]
    w = w_ref[...]
    b = b_ref[...]
    ye = jnp.dot(x2[:, :C], w, preferred_element_type=jnp.float32) + b
    yo = jnp.dot(x2[:, C:], w, preferred_element_type=jnp.float32) + b
    o_ref[...] = jnp.maximum(ye, yo)


def _down_pool(x, w, b):
    M, C = x.shape
    C2 = w.shape[1]
    x2 = x.reshape(M // 2, 2 * C)
    return pl.pallas_call(
        _down_body,
        out_shape=jax.ShapeDtypeStruct((M // 2, C2), jnp.float32),
    )(x2, w, b[None, :])


def _up_body(pa_ref, skip2_ref, upw, upb, skw, skb, o_ref):
    Cs = skw.shape[0]
    z = jnp.dot(pa_ref[...], upw[...], preferred_element_type=jnp.float32) + upb[...]
    s2 = skip2_ref[...]
    se = jnp.dot(s2[:, :Cs], skw[...], preferred_element_type=jnp.float32) + skb[...]
    so = jnp.dot(s2[:, Cs:], skw[...], preferred_element_type=jnp.float32) + skb[...]
    o_ref[...] = jnp.concatenate([z + se, z + so], axis=1)


def _up_skip(parent_inv, skip, upw, upb, skw, skb):
    M, Cs = skip.shape
    Co = upw.shape[1]
    skip2 = skip.reshape(M // 2, 2 * Cs)
    out2 = pl.pallas_call(
        _up_body,
        out_shape=jax.ShapeDtypeStruct((M // 2, 2 * Co), jnp.float32),
    )(parent_inv, skip2, upw, upb[None, :], skw, skb[None, :])
    return out2.reshape(M, Co)


def kernel(points, params):
    flat = points.reshape(_B * _N, 3)
    orders = _all_orders(flat)

    x = flat @ params['embed_w'] + params['embed_b']
    skips = []
    for s in range(5):
        x = _sc_gather(x, orders[s])
        for bp in params['enc'][s]['blocks']:
            x = _attn_block(x, bp, _ENC_H[s])
        skips.append(x)
        if s < 4:
            sp = params['enc'][s]
            x = _down_pool(x, sp['down_w'], sp['down_b'])

    for s in range(3, -1, -1):
        dp = params['dec'][s]
        parent_inv = _sc_gather(x, _inv_perm(orders[s + 1]))
        x = _up_skip(parent_inv, skips[s], dp['up_w'], dp['up_b'],
                     dp['skip_w'], dp['skip_b'])
        for bp in dp['blocks']:
            x = _attn_block(x, bp, _DEC_H[s])

    x = _sc_gather(x, _inv_perm(orders[0]))
    per_point = x.reshape(_B, _N, _DEC_CH[0])
    global_feat = jnp.max(per_point, axis=1)
    return per_point, global_feat


# speedup vs baseline: 1.0065x; 1.0065x over previous
import functools

import jax
import jax.numpy as jnp
from jax import lax
from jax.experimental import pallas as pl
from jax.experimental.pallas import tpu as pltpu
from jax.experimental.pallas import tpu_sc as plsc

_B = 1
_N = 16384
_PATCH = 1024
_GRID = 0.02
_ENC_DEPTHS = (2, 2, 2, 4, 2)
_ENC_CH = (32, 64, 128, 256, 384)
_ENC_H = (2, 4, 8, 16, 24)
_DEC_DEPTHS = (2, 2, 2, 2)
_DEC_CH = (64, 64, 128, 256)
_DEC_H = (4, 4, 8, 16)


def _split3(a):
    a = a & jnp.uint32(0x3FF)
    a = (a | (a << 16)) & jnp.uint32(0x030000FF)
    a = (a | (a << 8)) & jnp.uint32(0x0300F00F)
    a = (a | (a << 4)) & jnp.uint32(0x030C30C3)
    a = (a | (a << 2)) & jnp.uint32(0x09249249)
    return a


def _morton(coord, grid):
    g = jnp.floor(coord / grid).astype(jnp.uint32)
    return _split3(g[:, 0]) | (_split3(g[:, 1]) << 1) | (_split3(g[:, 2]) << 2)


def _all_orders(flat):
    coord = flat
    orders = []
    for s in range(5):
        code = _morton(coord, _GRID * (2 ** s))
        order = jnp.argsort(code)
        coord = coord[order]
        orders.append(order)
        if s < 4:
            coord = jnp.mean(coord.reshape(-1, 2, 3), axis=1)
    return orders


def _inv_perm(order):
    return jnp.argsort(order)


_SC_NC = 2
_SC_NS = 16
_SC_NW = _SC_NC * _SC_NS


def _sc_gather(x, idx):
    M, C = x.shape
    if C % 128 != 0:
        return x[idx]
    bpw = M // _SC_NW
    chunk = min(bpw, 128)
    nchunk = bpw // chunk
    mesh = plsc.VectorSubcoreMesh(core_axis_name="c", subcore_axis_name="s")

    @functools.partial(
        pl.kernel, mesh=mesh,
        out_type=jax.ShapeDtypeStruct((M, C), jnp.float32),
        scratch_types=[pltpu.VMEM((bpw,), jnp.int32),
                       pltpu.VMEM((bpw, C), jnp.float32),
                       pltpu.SemaphoreType.DMA])
    def gk(x_hbm, idx_hbm, out_hbm, idx_v, rows_v, sem):
        wid = lax.axis_index("s") * _SC_NC + lax.axis_index("c")
        base = wid * bpw
        pltpu.sync_copy(idx_hbm.at[pl.ds(base, bpw)], idx_v)
        copies = [
            pltpu.async_copy(
                x_hbm.at[idx_v.at[pl.ds(j * chunk, chunk)]],
                rows_v.at[pl.ds(j * chunk, chunk)], sem)
            for j in range(nchunk)
        ]
        for c in copies:
            c.wait()
        pltpu.sync_copy(rows_v, out_hbm.at[pl.ds(base, bpw)])

    return gk(x, idx.astype(jnp.int32))


def _ln(x, s, b):
    mu = jnp.mean(x, axis=-1, keepdims=True)
    var = jnp.mean((x - mu) ** 2, axis=-1, keepdims=True)
    return (x - mu) / jnp.sqrt(var + 1e-5) * s + b


def _one_block(x, nheads, ln1s, ln1b, qkvw, qkvb, projw, projb,
               ln2s, ln2b, w1, b1, w2, b2):
    C = x.shape[1]
    d = C // nheads
    h = _ln(x, ln1s[...], ln1b[...])
    qkv = jnp.dot(h, qkvw[...], preferred_element_type=jnp.float32) + qkvb[...]
    scale = d ** -0.5
    outs = []
    qkv16 = qkv.astype(jnp.bfloat16)
    ones = jnp.ones((x.shape[0], 1), jnp.bfloat16)
    for hd in range(nheads):
        q = (qkv[:, hd * d:(hd + 1) * d] * scale).astype(jnp.bfloat16)
        k = qkv16[:, C + hd * d:C + (hd + 1) * d]
        v = qkv16[:, 2 * C + hd * d:2 * C + (hd + 1) * d]
        s = jax.lax.dot_general(q, k, (((1,), (1,)), ((), ())),
                                preferred_element_type=jnp.float32)
        e = jnp.exp(s.astype(jnp.bfloat16))
        ov = jnp.dot(e, jnp.concatenate([v, ones], axis=1),
                     preferred_element_type=jnp.float32)
        outs.append(ov[:, :d] / ov[:, d:d + 1])
    o = jnp.concatenate(outs, axis=1)
    x = x + jnp.dot(o, projw[...], preferred_element_type=jnp.float32) + projb[...]
    h = _ln(x, ln2s[...], ln2b[...])
    h = jax.nn.gelu(jnp.dot(h, w1[...], preferred_element_type=jnp.float32) + b1[...])
    h = jnp.dot(h, w2[...], preferred_element_type=jnp.float32) + b2[...]
    return x + h


def _stage_body(nheads, depth, *refs):
    x_ref = refs[0]
    o_ref = refs[-1]
    wrefs = refs[1:-1]
    x = x_ref[...]
    for b in range(depth):
        x = _one_block(x, nheads, *wrefs[12 * b:12 * (b + 1)])
    o_ref[...] = x


def _flat_ws(p):
    return [p['ln1_s'][None, :], p['ln1_b'][None, :], p['qkv_w'], p['qkv_b'][None, :],
            p['proj_w'], p['proj_b'][None, :], p['ln2_s'][None, :], p['ln2_b'][None, :],
            p['mlp_w1'], p['mlp_b1'][None, :], p['mlp_w2'], p['mlp_b2'][None, :]]


def _attn_stage(x, block_params, nheads):
    M, C = x.shape
    ws = []
    for p in block_params:
        ws += _flat_ws(p)
    grid = (M // _PATCH,)
    in_specs = [pl.BlockSpec((_PATCH, C), lambda i: (i, 0))]
    in_specs += [pl.BlockSpec(w.shape, lambda i: (0, 0)) for w in ws]
    return pl.pallas_call(
        functools.partial(_stage_body, nheads, len(block_params)),
        grid=grid,
        in_specs=in_specs,
        out_specs=pl.BlockSpec((_PATCH, C), lambda i: (i, 0)),
        out_shape=jax.ShapeDtypeStruct((M, C), jnp.float32),
        compiler_params=pltpu.CompilerParams(
            dimension_semantics=("parallel",)),
    )(x, *ws)


def _down_body(x2_ref, w_ref, b_ref, o_ref):
    C = w_ref.shape[0]
    x2 = x2_ref[...]
    w = w_ref[...]
    b = b_ref[...]
    ye = jnp.dot(x2[:, :C], w, preferred_element_type=jnp.float32) + b
    yo = jnp.dot(x2[:, C:], w, preferred_element_type=jnp.float32) + b
    o_ref[...] = jnp.maximum(ye, yo)


def _down_pool(x, w, b):
    M, C = x.shape
    C2 = w.shape[1]
    x2 = x.reshape(M // 2, 2 * C)
    return pl.pallas_call(
        _down_body,
        out_shape=jax.ShapeDtypeStruct((M // 2, C2), jnp.float32),
    )(x2, w, b[None, :])


def _up_body(pa_ref, skip2_ref, upw, upb, skw, skb, o_ref):
    Cs = skw.shape[0]
    z = jnp.dot(pa_ref[...], upw[...], preferred_element_type=jnp.float32) + upb[...]
    s2 = skip2_ref[...]
    se = jnp.dot(s2[:, :Cs], skw[...], preferred_element_type=jnp.float32) + skb[...]
    so = jnp.dot(s2[:, Cs:], skw[...], preferred_element_type=jnp.float32) + skb[...]
    o_ref[...] = jnp.concatenate([z + se, z + so], axis=1)


def _up_skip(parent_inv, skip, upw, upb, skw, skb):
    M, Cs = skip.shape
    Co = upw.shape[1]
    skip2 = skip.reshape(M // 2, 2 * Cs)
    out2 = pl.pallas_call(
        _up_body,
        out_shape=jax.ShapeDtypeStruct((M // 2, 2 * Co), jnp.float32),
    )(parent_inv, skip2, upw, upb[None, :], skw, skb[None, :])
    return out2.reshape(M, Co)


def kernel(points, params):
    flat = points.reshape(_B * _N, 3)
    orders = _all_orders(flat)

    x = flat @ params['embed_w'] + params['embed_b']
    skips = []
    for s in range(5):
        x = _sc_gather(x, orders[s])
        x = _attn_stage(x, params['enc'][s]['blocks'], _ENC_H[s])
        skips.append(x)
        if s < 4:
            sp = params['enc'][s]
            x = _down_pool(x, sp['down_w'], sp['down_b'])

    for s in range(3, -1, -1):
        dp = params['dec'][s]
        parent_inv = _sc_gather(x, _inv_perm(orders[s + 1]))
        x = _up_skip(parent_inv, skips[s], dp['up_w'], dp['up_b'],
                     dp['skip_w'], dp['skip_b'])
        x = _attn_stage(x, dp['blocks'], _DEC_H[s])

    x = _sc_gather(x, _inv_perm(orders[0]))
    per_point = x.reshape(_B, _N, _DEC_CH[0])
    global_feat = jnp.max(per_point, axis=1)
    return per_point, global_feat
